# Initial kernel scaffold; baseline (speedup 1.0000x reference)
import jax
import jax.numpy as jnp
from jax.experimental import pallas as pl


def kernel(distances, vec, angle_src, angle_dst):
    raise NotImplementedError("write your pallas kernel here")


# speedup vs baseline: 397.2060x; 397.2060x over previous
import functools

import jax
import jax.numpy as jnp
from jax import lax
from jax.experimental import pallas as pl
from jax.experimental.pallas import tpu as pltpu
from jax.experimental.pallas import tpu_sc as plsc

NC = 2
NS = 16
LANES = 16
CHUNK = 2000
TBLK = 1000


def _sc_angle_kernel(E: int, A: int):
    n_workers = NC * NS
    per_w = A // n_workers
    n_chunks = per_w // CHUNK
    n_tb = E // (NS * TBLK)
    assert per_w * n_workers == A and n_chunks * CHUNK == per_w
    assert n_tb * NS * TBLK == E and n_chunks % 2 == 0

    mesh = plsc.VectorSubcoreMesh(
        core_axis_name="c", subcore_axis_name="s",
        num_cores=NC, num_subcores=NS)

    @functools.partial(
        pl.kernel,
        out_type=jax.ShapeDtypeStruct((A,), jnp.float32),
        mesh=mesh,
        scratch_types=[
            pltpu.VMEM_SHARED((E, 4), jnp.float32),
            pltpu.VMEM((TBLK,), jnp.float32),
            pltpu.VMEM((TBLK,), jnp.float32),
            pltpu.VMEM((TBLK,), jnp.float32),
            pltpu.VMEM((TBLK,), jnp.float32),
            pltpu.VMEM((TBLK, 4), jnp.float32),
            pltpu.VMEM((2, CHUNK), jnp.int32),
            pltpu.VMEM((2, CHUNK), jnp.int32),
            pltpu.VMEM((2, CHUNK, 4), jnp.float32),
            pltpu.VMEM((2, CHUNK, 4), jnp.float32),
            pltpu.VMEM((2, CHUNK), jnp.float32),
            pltpu.SemaphoreType.DMA,
            pltpu.SemaphoreType.DMA,
            pltpu.SemaphoreType.DMA,
            pltpu.SemaphoreType.DMA,
        ],
        compiler_params=pltpu.CompilerParams(
            needs_layout_passes=False, use_tc_tiling_on_sc=False),
    )
    def sc_angles(d_hbm, vx_hbm, vy_hbm, vz_hbm, src_hbm, dst_hbm, out_hbm,
                  table, bx, by, bz, bd, btile,
                  sidx, didx, srow, drow, outv, g0, g1, o0, o1):
        cid = lax.axis_index("c")
        sid = lax.axis_index("s")
        wid = sid * NC + cid
        base_w = wid * per_w
        gsem = (g0, g1)
        osem = (o0, o1)

        c0 = jnp.zeros((LANES,), jnp.int32)
        c1 = jnp.full((LANES,), 1, jnp.int32)
        c2 = jnp.full((LANES,), 2, jnp.int32)
        c3 = jnp.full((LANES,), 3, jnp.int32)

        def build_step(tb, carry):
            base = (sid * n_tb + tb) * TBLK
            pltpu.sync_copy(vx_hbm.at[pl.ds(base, TBLK)], bx)
            pltpu.sync_copy(vy_hbm.at[pl.ds(base, TBLK)], by)
            pltpu.sync_copy(vz_hbm.at[pl.ds(base, TBLK)], bz)
            pltpu.sync_copy(d_hbm.at[pl.ds(base, TBLK)], bd)

            def inter_body(i, c):
                r = lax.iota(jnp.int32, LANES) + i * LANES
                sl = pl.ds(i * LANES, LANES)
                plsc.store_scatter(btile, [r, c0], bx[sl])
                plsc.store_scatter(btile, [r, c1], by[sl])
                plsc.store_scatter(btile, [r, c2], bz[sl])
                plsc.store_scatter(btile, [r, c3], bd[sl])
                return c

            lax.fori_loop(0, TBLK // LANES, inter_body, 0, unroll=4)
            pltpu.sync_copy(btile, table.at[pl.ds(base, TBLK)])
            return carry

        lax.fori_loop(0, n_tb, build_step, 0)
        plsc.subcore_barrier()

        def gather_start(k, b):
            base = base_w + k * CHUNK
            pltpu.sync_copy(src_hbm.at[pl.ds(base, CHUNK)], sidx.at[b])
            pltpu.sync_copy(dst_hbm.at[pl.ds(base, CHUNK)], didx.at[b])
            pltpu.async_copy(table.at[sidx.at[b]], srow.at[b], gsem[b])
            pltpu.async_copy(table.at[didx.at[b]], drow.at[b], gsem[b])

        def gather_wait(b):
            pltpu.make_async_copy(
                table.at[sidx.at[b]], srow.at[b], gsem[b]).wait()
            pltpu.make_async_copy(
                table.at[didx.at[b]], drow.at[b], gsem[b]).wait()

        def compute(k, b):
            sr = srow.at[b]
            dr = drow.at[b]
            ov = outv.at[b]

            def vec_body(i, carry2):
                r = lax.iota(jnp.int32, LANES) + i * LANES
                sx = plsc.load_gather(sr, [r, c0])
                sy = plsc.load_gather(sr, [r, c1])
                sz = plsc.load_gather(sr, [r, c2])
                sd = plsc.load_gather(sr, [r, c3])
                dx = plsc.load_gather(dr, [r, c0])
                dy = plsc.load_gather(dr, [r, c1])
                dz = plsc.load_gather(dr, [r, c2])
                dd = plsc.load_gather(dr, [r, c3])
                num = sx * dx + sy * dy + sz * dz
                den = jnp.maximum(sd * dd, jnp.float32(1e-10))
                u = jnp.float32(0.95) * (num / den)
                au = jnp.abs(u)
                t = jnp.float32(1.0) - au
                yi = lax.bitcast_convert_type(t, jnp.int32)
                yi = jnp.int32(0x5F3759DF) - lax.shift_right_arithmetic(
                    yi, jnp.int32(1))
                y = lax.bitcast_convert_type(yi, jnp.float32)
                half_t = jnp.float32(0.5) * t
                for _ in range(3):
                    y = y * (jnp.float32(1.5) - half_t * y * y)
                s = t * y
                p = jnp.float32(-0.0012624911)
                for coef in (0.0066700901, -0.0170881256, 0.0308918810,
                             -0.0501743046, 0.0889789874, -0.2145988016,
                             1.5707963050):
                    p = p * au + jnp.float32(coef)
                pos = s * p
                ov[pl.ds(i * LANES, LANES)] = jnp.where(
                    u >= 0, pos, jnp.float32(3.14159265359) - pos)
                return carry2

            lax.fori_loop(0, CHUNK // LANES, vec_body, 0, unroll=4)
            pltpu.async_copy(
                ov, out_hbm.at[pl.ds(base_w + k * CHUNK, CHUNK)], osem[b])

        def out_wait(k, b):
            pltpu.make_async_copy(
                outv.at[b], out_hbm.at[pl.ds(base_w + k * CHUNK, CHUNK)],
                osem[b]).wait()

        gather_start(0, 0)
        gather_start(1, 1)

        def pair_body(t, carry):
            k0 = 2 * t
            for b in (0, 1):
                k = k0 + b
                gather_wait(b)

                @pl.when(t > 0)
                def _():
                    out_wait(k, b)

                compute(k, b)

                @pl.when(k + 2 < n_chunks)
                def _():
                    gather_start(k + 2, b)
            return carry

        lax.fori_loop(0, n_chunks // 2, pair_body, 0)
        out_wait(n_chunks - 2, 0)
        out_wait(n_chunks - 1, 1)

    return sc_angles


def kernel(distances, vec, angle_src, angle_dst):
    A = angle_src.shape[0]
    E = distances.shape[0]
    vx = vec[:, 0]
    vy = vec[:, 1]
    vz = vec[:, 2]
    return _sc_angle_kernel(E, A)(
        distances, vx, vy, vz, angle_src, angle_dst)


# speedup vs baseline: 397.4494x; 1.0006x over previous
import functools

import jax
import jax.numpy as jnp
from jax import lax
from jax.experimental import pallas as pl
from jax.experimental.pallas import tpu as pltpu
from jax.experimental.pallas import tpu_sc as plsc

NC = 2
NS = 16
LANES = 16
CHUNK = 2000
TBLK = 1000


def _sc_angle_kernel(E: int, A: int):
    n_workers = NC * NS
    per_w = A // n_workers
    n_chunks = per_w // CHUNK
    n_tb = E // (NS * TBLK)
    assert per_w * n_workers == A and n_chunks * CHUNK == per_w
    assert n_tb * NS * TBLK == E and n_chunks % 2 == 0

    mesh = plsc.VectorSubcoreMesh(
        core_axis_name="c", subcore_axis_name="s",
        num_cores=NC, num_subcores=NS)

    @functools.partial(
        pl.kernel,
        out_type=jax.ShapeDtypeStruct((A,), jnp.float32),
        mesh=mesh,
        scratch_types=[
            pltpu.VMEM_SHARED((E, 4), jnp.float32),
            pltpu.VMEM((TBLK,), jnp.float32),
            pltpu.VMEM((TBLK,), jnp.float32),
            pltpu.VMEM((TBLK,), jnp.float32),
            pltpu.VMEM((TBLK,), jnp.float32),
            pltpu.VMEM((TBLK, 4), jnp.float32),
            pltpu.VMEM((2, CHUNK), jnp.int32),
            pltpu.VMEM((2, CHUNK), jnp.int32),
            pltpu.VMEM((2, CHUNK, 4), jnp.float32),
            pltpu.VMEM((2, CHUNK, 4), jnp.float32),
            pltpu.VMEM((2, CHUNK), jnp.float32),
            pltpu.SemaphoreType.DMA,
            pltpu.SemaphoreType.DMA,
            pltpu.SemaphoreType.DMA,
            pltpu.SemaphoreType.DMA,
        ],
        compiler_params=pltpu.CompilerParams(
            needs_layout_passes=False, use_tc_tiling_on_sc=False),
    )
    def sc_angles(d_hbm, vx_hbm, vy_hbm, vz_hbm, src_hbm, dst_hbm, out_hbm,
                  table, bx, by, bz, bd, btile,
                  sidx, didx, srow, drow, outv, g0, g1, o0, o1):
        cid = lax.axis_index("c")
        sid = lax.axis_index("s")
        wid = sid * NC + cid
        base_w = wid * per_w
        gsem = (g0, g1)
        osem = (o0, o1)

        c0 = jnp.zeros((LANES,), jnp.int32)
        c1 = jnp.full((LANES,), 1, jnp.int32)
        c2 = jnp.full((LANES,), 2, jnp.int32)
        c3 = jnp.full((LANES,), 3, jnp.int32)

        def build_step(tb, carry):
            base = (sid * n_tb + tb) * TBLK
            pltpu.sync_copy(vx_hbm.at[pl.ds(base, TBLK)], bx)
            pltpu.sync_copy(vy_hbm.at[pl.ds(base, TBLK)], by)
            pltpu.sync_copy(vz_hbm.at[pl.ds(base, TBLK)], bz)
            pltpu.sync_copy(d_hbm.at[pl.ds(base, TBLK)], bd)

            def inter_body(i, c):
                r = lax.iota(jnp.int32, LANES) + i * LANES
                sl = pl.ds(i * LANES, LANES)
                plsc.store_scatter(btile, [r, c0], bx[sl])
                plsc.store_scatter(btile, [r, c1], by[sl])
                plsc.store_scatter(btile, [r, c2], bz[sl])
                plsc.store_scatter(btile, [r, c3], bd[sl])
                return c

            lax.fori_loop(0, TBLK // LANES, inter_body, 0, unroll=4)
            pltpu.sync_copy(btile, table.at[pl.ds(base, TBLK)])
            return carry

        with jax.named_scope("table_build"):
            lax.fori_loop(0, n_tb, build_step, 0)
            plsc.subcore_barrier()

        def gather_start(k, b):
            base = base_w + k * CHUNK
            pltpu.sync_copy(src_hbm.at[pl.ds(base, CHUNK)], sidx.at[b])
            pltpu.sync_copy(dst_hbm.at[pl.ds(base, CHUNK)], didx.at[b])
            pltpu.async_copy(table.at[sidx.at[b]], srow.at[b], gsem[b])
            pltpu.async_copy(table.at[didx.at[b]], drow.at[b], gsem[b])

        def gather_wait(b):
            pltpu.make_async_copy(
                table.at[sidx.at[b]], srow.at[b], gsem[b]).wait()
            pltpu.make_async_copy(
                table.at[didx.at[b]], drow.at[b], gsem[b]).wait()

        def compute(k, b):
            sr = srow.at[b]
            dr = drow.at[b]
            ov = outv.at[b]

            def vec_body(i, carry2):
                r = lax.iota(jnp.int32, LANES) + i * LANES
                sx = plsc.load_gather(sr, [r, c0])
                sy = plsc.load_gather(sr, [r, c1])
                sz = plsc.load_gather(sr, [r, c2])
                sd = plsc.load_gather(sr, [r, c3])
                dx = plsc.load_gather(dr, [r, c0])
                dy = plsc.load_gather(dr, [r, c1])
                dz = plsc.load_gather(dr, [r, c2])
                dd = plsc.load_gather(dr, [r, c3])
                num = sx * dx + sy * dy + sz * dz
                den = jnp.maximum(sd * dd, jnp.float32(1e-10))
                u = jnp.float32(0.95) * (num / den)
                au = jnp.abs(u)
                t = jnp.float32(1.0) - au
                yi = lax.bitcast_convert_type(t, jnp.int32)
                yi = jnp.int32(0x5F3759DF) - lax.shift_right_arithmetic(
                    yi, jnp.int32(1))
                y = lax.bitcast_convert_type(yi, jnp.float32)
                half_t = jnp.float32(0.5) * t
                for _ in range(3):
                    y = y * (jnp.float32(1.5) - half_t * y * y)
                s = t * y
                p = jnp.float32(-0.0012624911)
                for coef in (0.0066700901, -0.0170881256, 0.0308918810,
                             -0.0501743046, 0.0889789874, -0.2145988016,
                             1.5707963050):
                    p = p * au + jnp.float32(coef)
                pos = s * p
                ov[pl.ds(i * LANES, LANES)] = jnp.where(
                    u >= 0, pos, jnp.float32(3.14159265359) - pos)
                return carry2

            lax.fori_loop(0, CHUNK // LANES, vec_body, 0, unroll=4)
            pltpu.async_copy(
                ov, out_hbm.at[pl.ds(base_w + k * CHUNK, CHUNK)], osem[b])

        def out_wait(k, b):
            pltpu.make_async_copy(
                outv.at[b], out_hbm.at[pl.ds(base_w + k * CHUNK, CHUNK)],
                osem[b]).wait()

        with jax.named_scope("prologue"):
            gather_start(0, 0)
            gather_start(1, 1)

        def pair_body(t, carry):
            k0 = 2 * t
            for b in (0, 1):
                k = k0 + b
                gather_wait(b)

                @pl.when(t > 0)
                def _():
                    out_wait(k, b)

                compute(k, b)

                @pl.when(k + 2 < n_chunks)
                def _():
                    gather_start(k + 2, b)
            return carry

        lax.fori_loop(0, n_chunks // 2, pair_body, 0)
        out_wait(n_chunks - 2, 0)
        out_wait(n_chunks - 1, 1)

    return sc_angles


def kernel(distances, vec, angle_src, angle_dst):
    A = angle_src.shape[0]
    E = distances.shape[0]
    vx = vec[:, 0]
    vy = vec[:, 1]
    vz = vec[:, 2]
    return _sc_angle_kernel(E, A)(
        distances, vx, vy, vz, angle_src, angle_dst)


# speedup vs baseline: 448.4633x; 1.1284x over previous
import functools

import jax
import jax.numpy as jnp
from jax import lax
from jax.experimental import pallas as pl
from jax.experimental.pallas import tpu as pltpu
from jax.experimental.pallas import tpu_sc as plsc

NC = 2
NS = 16
LANES = 16
CHUNK = 2000
TBLK = 1000


def _sc_angle_kernel(E: int, A: int):
    n_workers = NC * NS
    per_w = A // n_workers
    n_chunks = per_w // CHUNK
    n_tb = E // (NS * TBLK)
    assert per_w * n_workers == A and n_chunks * CHUNK == per_w
    assert n_tb * NS * TBLK == E and n_chunks % 2 == 0

    mesh = plsc.VectorSubcoreMesh(
        core_axis_name="c", subcore_axis_name="s",
        num_cores=NC, num_subcores=NS)

    @functools.partial(
        pl.kernel,
        out_type=jax.ShapeDtypeStruct((A,), jnp.float32),
        mesh=mesh,
        scratch_types=[
            pltpu.VMEM_SHARED((E, 4), jnp.float32),
            pltpu.VMEM((TBLK,), jnp.float32),
            pltpu.VMEM((TBLK,), jnp.float32),
            pltpu.VMEM((TBLK,), jnp.float32),
            pltpu.VMEM((TBLK,), jnp.float32),
            pltpu.VMEM((TBLK, 4), jnp.float32),
            pltpu.VMEM((2, CHUNK), jnp.int32),
            pltpu.VMEM((2, CHUNK), jnp.int32),
            pltpu.VMEM((2, CHUNK, 4), jnp.float32),
            pltpu.VMEM((2, CHUNK, 4), jnp.float32),
            pltpu.VMEM((2, CHUNK), jnp.float32),
            pltpu.SemaphoreType.DMA,
            pltpu.SemaphoreType.DMA,
            pltpu.SemaphoreType.DMA,
            pltpu.SemaphoreType.DMA,
        ],
        compiler_params=pltpu.CompilerParams(
            needs_layout_passes=False, use_tc_tiling_on_sc=False),
    )
    def sc_angles(d_hbm, vx_hbm, vy_hbm, vz_hbm, src_hbm, dst_hbm, out_hbm,
                  table, bx, by, bz, bd, btile,
                  sidx, didx, srow, drow, outv, g0, g1, o0, o1):
        cid = lax.axis_index("c")
        sid = lax.axis_index("s")
        wid = sid * NC + cid
        base_w = wid * per_w
        gsem = (g0, g1)
        osem = (o0, o1)

        c0 = jnp.zeros((LANES,), jnp.int32)
        c1 = jnp.full((LANES,), 1, jnp.int32)
        c2 = jnp.full((LANES,), 2, jnp.int32)
        c3 = jnp.full((LANES,), 3, jnp.int32)

        def build_step(tb, carry):
            base = (sid * n_tb + tb) * TBLK
            sl = pl.ds(base, TBLK)
            pltpu.async_copy(vx_hbm.at[sl], bx, g0)
            pltpu.async_copy(vy_hbm.at[sl], by, g0)
            pltpu.async_copy(vz_hbm.at[sl], bz, g0)
            pltpu.async_copy(d_hbm.at[sl], bd, g0)
            pltpu.make_async_copy(vx_hbm.at[sl], bx, g0).wait()
            pltpu.make_async_copy(vy_hbm.at[sl], by, g0).wait()
            pltpu.make_async_copy(vz_hbm.at[sl], bz, g0).wait()
            pltpu.make_async_copy(d_hbm.at[sl], bd, g0).wait()

            def inter_body(i, c):
                r = lax.iota(jnp.int32, LANES) + i * LANES
                sl = pl.ds(i * LANES, LANES)
                plsc.store_scatter(btile, [r, c0], bx[sl])
                plsc.store_scatter(btile, [r, c1], by[sl])
                plsc.store_scatter(btile, [r, c2], bz[sl])
                plsc.store_scatter(btile, [r, c3], bd[sl])
                return c

            lax.fori_loop(0, TBLK // LANES, inter_body, 0, unroll=4)
            pltpu.sync_copy(btile, table.at[pl.ds(base, TBLK)])
            return carry

        with jax.named_scope("table_build"):
            lax.fori_loop(0, n_tb, build_step, 0)
            plsc.subcore_barrier()

        def gather_start(k, b):
            base = base_w + k * CHUNK
            pltpu.sync_copy(src_hbm.at[pl.ds(base, CHUNK)], sidx.at[b])
            pltpu.sync_copy(dst_hbm.at[pl.ds(base, CHUNK)], didx.at[b])
            pltpu.async_copy(table.at[sidx.at[b]], srow.at[b], gsem[b])
            pltpu.async_copy(table.at[didx.at[b]], drow.at[b], gsem[b])

        def gather_wait(b):
            pltpu.make_async_copy(
                table.at[sidx.at[b]], srow.at[b], gsem[b]).wait()
            pltpu.make_async_copy(
                table.at[didx.at[b]], drow.at[b], gsem[b]).wait()

        def compute(k, b):
            sr = srow.at[b]
            dr = drow.at[b]
            ov = outv.at[b]

            def vec_body(i, carry2):
                r = lax.iota(jnp.int32, LANES) + i * LANES
                sx = plsc.load_gather(sr, [r, c0])
                sy = plsc.load_gather(sr, [r, c1])
                sz = plsc.load_gather(sr, [r, c2])
                sd = plsc.load_gather(sr, [r, c3])
                dx = plsc.load_gather(dr, [r, c0])
                dy = plsc.load_gather(dr, [r, c1])
                dz = plsc.load_gather(dr, [r, c2])
                dd = plsc.load_gather(dr, [r, c3])
                num = sx * dx + sy * dy + sz * dz
                den = jnp.maximum(sd * dd, jnp.float32(1e-10))
                u = jnp.float32(0.95) * (num / den)
                au = jnp.abs(u)
                t = jnp.float32(1.0) - au
                yi = lax.bitcast_convert_type(t, jnp.int32)
                yi = jnp.int32(0x5F3759DF) - lax.shift_right_arithmetic(
                    yi, jnp.int32(1))
                y = lax.bitcast_convert_type(yi, jnp.float32)
                half_t = jnp.float32(0.5) * t
                for _ in range(2):
                    y = y * (jnp.float32(1.5) - half_t * y * y)
                s = t * y
                p = jnp.float32(-0.0187293)
                for coef in (0.0742610, -0.2121144, 1.5707288):
                    p = p * au + jnp.float32(coef)
                pos = s * p
                ov[pl.ds(i * LANES, LANES)] = jnp.where(
                    u >= 0, pos, jnp.float32(3.14159265359) - pos)
                return carry2

            lax.fori_loop(0, CHUNK // LANES, vec_body, 0, unroll=4)
            pltpu.async_copy(
                ov, out_hbm.at[pl.ds(base_w + k * CHUNK, CHUNK)], osem[b])

        def out_wait(k, b):
            pltpu.make_async_copy(
                outv.at[b], out_hbm.at[pl.ds(base_w + k * CHUNK, CHUNK)],
                osem[b]).wait()

        with jax.named_scope("prologue"):
            gather_start(0, 0)
            gather_start(1, 1)

        def pair_body(t, carry):
            k0 = 2 * t
            for b in (0, 1):
                k = k0 + b
                gather_wait(b)

                @pl.when(t > 0)
                def _():
                    out_wait(k, b)

                compute(k, b)

                @pl.when(k + 2 < n_chunks)
                def _():
                    gather_start(k + 2, b)
            return carry

        lax.fori_loop(0, n_chunks // 2, pair_body, 0)
        out_wait(n_chunks - 2, 0)
        out_wait(n_chunks - 1, 1)

    return sc_angles


def kernel(distances, vec, angle_src, angle_dst):
    A = angle_src.shape[0]
    E = distances.shape[0]
    vx = vec[:, 0]
    vy = vec[:, 1]
    vz = vec[:, 2]
    return _sc_angle_kernel(E, A)(
        distances, vx, vy, vz, angle_src, angle_dst)


# speedup vs baseline: 527.0626x; 1.1753x over previous
import functools

import jax
import jax.numpy as jnp
from jax import lax
from jax.experimental import pallas as pl
from jax.experimental.pallas import tpu as pltpu
from jax.experimental.pallas import tpu_sc as plsc

NC = 2
NS = 16
LANES = 16
CHUNK = 2000
TBLK = 1000


def _sc_angle_kernel(E: int, A: int):
    n_workers = NC * NS
    per_w = A // n_workers
    n_chunks = per_w // CHUNK
    n_tb = E // (NS * TBLK)
    assert per_w * n_workers == A and n_chunks * CHUNK == per_w
    assert n_tb * NS * TBLK == E and n_chunks % 2 == 0

    mesh = plsc.VectorSubcoreMesh(
        core_axis_name="c", subcore_axis_name="s",
        num_cores=NC, num_subcores=NS)

    @functools.partial(
        pl.kernel,
        out_type=jax.ShapeDtypeStruct((A,), jnp.float32),
        mesh=mesh,
        scratch_types=[
            pltpu.VMEM_SHARED((E, 3), jnp.float32),
            pltpu.VMEM((TBLK,), jnp.float32),
            pltpu.VMEM((TBLK,), jnp.float32),
            pltpu.VMEM((TBLK,), jnp.float32),
            pltpu.VMEM((TBLK,), jnp.float32),
            pltpu.VMEM((TBLK, 3), jnp.float32),
            pltpu.VMEM((2, CHUNK), jnp.int32),
            pltpu.VMEM((2, CHUNK), jnp.int32),
            pltpu.VMEM((2, CHUNK, 3), jnp.float32),
            pltpu.VMEM((2, CHUNK, 3), jnp.float32),
            pltpu.VMEM((2, CHUNK), jnp.float32),
            pltpu.SemaphoreType.DMA,
            pltpu.SemaphoreType.DMA,
            pltpu.SemaphoreType.DMA,
            pltpu.SemaphoreType.DMA,
        ],
        compiler_params=pltpu.CompilerParams(
            needs_layout_passes=False, use_tc_tiling_on_sc=False),
    )
    def sc_angles(d_hbm, vx_hbm, vy_hbm, vz_hbm, src_hbm, dst_hbm, out_hbm,
                  table, bx, by, bz, bd, btile,
                  sidx, didx, srow, drow, outv, g0, g1, o0, o1):
        cid = lax.axis_index("c")
        sid = lax.axis_index("s")
        wid = sid * NC + cid
        base_w = wid * per_w
        gsem = (g0, g1)
        osem = (o0, o1)

        c0 = jnp.zeros((LANES,), jnp.int32)
        c1 = jnp.full((LANES,), 1, jnp.int32)
        c2 = jnp.full((LANES,), 2, jnp.int32)

        def build_step(tb, carry):
            base = (sid * n_tb + tb) * TBLK
            sl = pl.ds(base, TBLK)
            pltpu.async_copy(vx_hbm.at[sl], bx, g0)
            pltpu.async_copy(vy_hbm.at[sl], by, g0)
            pltpu.async_copy(vz_hbm.at[sl], bz, g0)
            pltpu.async_copy(d_hbm.at[sl], bd, g0)
            pltpu.make_async_copy(vx_hbm.at[sl], bx, g0).wait()
            pltpu.make_async_copy(vy_hbm.at[sl], by, g0).wait()
            pltpu.make_async_copy(vz_hbm.at[sl], bz, g0).wait()
            pltpu.make_async_copy(d_hbm.at[sl], bd, g0).wait()

            def inter_body(i, c):
                r = lax.iota(jnp.int32, LANES) + i * LANES
                sl = pl.ds(i * LANES, LANES)
                inv = jnp.float32(1.0) / bd[sl]
                plsc.store_scatter(btile, [r, c0], bx[sl] * inv)
                plsc.store_scatter(btile, [r, c1], by[sl] * inv)
                plsc.store_scatter(btile, [r, c2], bz[sl] * inv)
                return c

            lax.fori_loop(0, TBLK // LANES, inter_body, 0, unroll=4)
            pltpu.sync_copy(btile, table.at[pl.ds(base, TBLK)])
            return carry

        with jax.named_scope("table_build"):
            lax.fori_loop(0, n_tb, build_step, 0)
            plsc.subcore_barrier()

        def gather_start(k, b):
            base = base_w + k * CHUNK
            pltpu.async_copy(src_hbm.at[pl.ds(base, CHUNK)], sidx.at[b], gsem[b])
            pltpu.async_copy(dst_hbm.at[pl.ds(base, CHUNK)], didx.at[b], gsem[b])
            pltpu.make_async_copy(
                src_hbm.at[pl.ds(base, CHUNK)], sidx.at[b], gsem[b]).wait()
            pltpu.make_async_copy(
                dst_hbm.at[pl.ds(base, CHUNK)], didx.at[b], gsem[b]).wait()
            pltpu.async_copy(table.at[sidx.at[b]], srow.at[b], gsem[b])
            pltpu.async_copy(table.at[didx.at[b]], drow.at[b], gsem[b])

        def gather_wait(b):
            pltpu.make_async_copy(
                table.at[sidx.at[b]], srow.at[b], gsem[b]).wait()
            pltpu.make_async_copy(
                table.at[didx.at[b]], drow.at[b], gsem[b]).wait()

        def compute(k, b):
            sr = srow.at[b]
            dr = drow.at[b]
            ov = outv.at[b]

            def vec_body(i, carry2):
                r = lax.iota(jnp.int32, LANES) + i * LANES
                sx = plsc.load_gather(sr, [r, c0])
                sy = plsc.load_gather(sr, [r, c1])
                sz = plsc.load_gather(sr, [r, c2])
                dx = plsc.load_gather(dr, [r, c0])
                dy = plsc.load_gather(dr, [r, c1])
                dz = plsc.load_gather(dr, [r, c2])
                u = jnp.float32(0.95) * (sx * dx + sy * dy + sz * dz)
                au = jnp.abs(u)
                t = jnp.float32(1.0) - au
                yi = lax.bitcast_convert_type(t, jnp.int32)
                yi = jnp.int32(0x5F3759DF) - lax.shift_right_arithmetic(
                    yi, jnp.int32(1))
                y = lax.bitcast_convert_type(yi, jnp.float32)
                half_t = jnp.float32(0.5) * t
                for _ in range(2):
                    y = y * (jnp.float32(1.5) - half_t * y * y)
                s = t * y
                p = jnp.float32(-0.0187293)
                for coef in (0.0742610, -0.2121144, 1.5707288):
                    p = p * au + jnp.float32(coef)
                pos = s * p
                ov[pl.ds(i * LANES, LANES)] = jnp.where(
                    u >= 0, pos, jnp.float32(3.14159265359) - pos)
                return carry2

            lax.fori_loop(0, CHUNK // LANES, vec_body, 0, unroll=4)
            pltpu.async_copy(
                ov, out_hbm.at[pl.ds(base_w + k * CHUNK, CHUNK)], osem[b])

        def out_wait(k, b):
            pltpu.make_async_copy(
                outv.at[b], out_hbm.at[pl.ds(base_w + k * CHUNK, CHUNK)],
                osem[b]).wait()

        with jax.named_scope("prologue"):
            gather_start(0, 0)
            gather_start(1, 1)

        def pair_body(t, carry):
            k0 = 2 * t
            for b in (0, 1):
                k = k0 + b
                gather_wait(b)

                @pl.when(t > 0)
                def _():
                    out_wait(k, b)

                compute(k, b)

                @pl.when(k + 2 < n_chunks)
                def _():
                    gather_start(k + 2, b)
            return carry

        lax.fori_loop(0, n_chunks // 2, pair_body, 0)
        out_wait(n_chunks - 2, 0)
        out_wait(n_chunks - 1, 1)

    return sc_angles


def kernel(distances, vec, angle_src, angle_dst):
    A = angle_src.shape[0]
    E = distances.shape[0]
    vx = vec[:, 0]
    vy = vec[:, 1]
    vz = vec[:, 2]
    return _sc_angle_kernel(E, A)(
        distances, vx, vy, vz, angle_src, angle_dst)
